# Initial kernel scaffold; baseline (speedup 1.0000x reference)
import jax
import jax.numpy as jnp
from jax.experimental import pallas as pl


def kernel(tokens, pos, token_table, pos_table):
    raise NotImplementedError("write your pallas kernel here")


# speedup vs baseline: 3.4442x; 3.4442x over previous
import functools

import jax
import jax.numpy as jnp
from jax import lax
from jax.experimental import pallas as pl
from jax.experimental.pallas import tpu as pltpu
from jax.experimental.pallas import tpu_sc as plsc

TOKEN_EMB = 64
POS_EMB = 64
OUT_D = TOKEN_EMB + POS_EMB

NUM_CORES = 2
NUM_SUBCORES = 16
NW = NUM_CORES * NUM_SUBCORES

IDX_MINOR = 128
K = 4
CHUNK = K * IDX_MINOR


def _make_kernel(n_total: int):
  per_w = n_total // NW
  n_iters = per_w // CHUNK
  assert per_w % CHUNK == 0

  mesh = plsc.VectorSubcoreMesh(
      core_axis_name="c", subcore_axis_name="s",
      num_cores=NUM_CORES, num_subcores=NUM_SUBCORES)

  @functools.partial(
      pl.kernel,
      out_type=jax.ShapeDtypeStruct((n_total, OUT_D), jnp.float32),
      mesh=mesh,
      compiler_params=pltpu.CompilerParams(use_tc_tiling_on_sc=False),
      scratch_types=[
          pltpu.VMEM((K, IDX_MINOR), jnp.int32),
          pltpu.VMEM((K, IDX_MINOR), jnp.int32),
          pltpu.VMEM((CHUNK, TOKEN_EMB), jnp.float32),
          pltpu.VMEM((CHUNK, POS_EMB), jnp.float32),
          pltpu.SemaphoreType.DMA,
      ],
  )
  def emb_kernel(tok_hbm, pos_hbm, tok_tab_hbm, pos_tab_hbm, out_hbm,
                 tok_idx_v, pos_idx_v, tok_rows_v, pos_rows_v, sem):
    wid = lax.axis_index("s") * NUM_CORES + lax.axis_index("c")
    w_base = wid * per_w

    def body(g, _):
      base = w_base + g * CHUNK
      copies = []
      for j in range(K):
        pltpu.sync_copy(
            tok_hbm.at[pl.ds(base + j * IDX_MINOR, IDX_MINOR)],
            tok_idx_v.at[j])
        pltpu.sync_copy(
            pos_hbm.at[pl.ds(base + j * IDX_MINOR, IDX_MINOR)],
            pos_idx_v.at[j])
      for j in range(K):
        copies.append(pltpu.async_copy(
            tok_tab_hbm.at[tok_idx_v.at[j]],
            tok_rows_v.at[pl.ds(j * IDX_MINOR, IDX_MINOR)], sem))
        copies.append(pltpu.async_copy(
            pos_tab_hbm.at[pos_idx_v.at[j]],
            pos_rows_v.at[pl.ds(j * IDX_MINOR, IDX_MINOR)], sem))
      for c in copies:
        c.wait()
      pltpu.sync_copy(
          tok_rows_v,
          out_hbm.at[pl.ds(base, CHUNK), pl.ds(0, TOKEN_EMB)])
      pltpu.sync_copy(
          pos_rows_v,
          out_hbm.at[pl.ds(base, CHUNK), pl.ds(TOKEN_EMB, POS_EMB)])
      return 0

    lax.fori_loop(0, n_iters, body, 0)

  return emb_kernel


@jax.jit
def kernel(tokens, pos, token_table, pos_table):
  B, L = tokens.shape
  n_total = B * L
  emb = _make_kernel(n_total)
  out = emb(tokens.reshape(n_total), pos.reshape(n_total),
            token_table, pos_table)
  return out.reshape(B, L, OUT_D)


# speedup vs baseline: 4.0932x; 1.1884x over previous
import functools

import jax
import jax.numpy as jnp
from jax import lax
from jax.experimental import pallas as pl
from jax.experimental.pallas import tpu as pltpu
from jax.experimental.pallas import tpu_sc as plsc

TOKEN_EMB = 64
POS_EMB = 64
OUT_D = TOKEN_EMB + POS_EMB

NUM_CORES = 2
NUM_SUBCORES = 16
NW = NUM_CORES * NUM_SUBCORES

IDX_MINOR = 128
K = 2
CHUNK = K * IDX_MINOR


def _make_kernel(n_total: int):
  per_w = n_total // NW
  n_iters = per_w // CHUNK
  idx_rows = per_w // IDX_MINOR
  assert per_w % CHUNK == 0 and n_iters % 2 == 0 and n_iters >= 4

  mesh = plsc.VectorSubcoreMesh(
      core_axis_name="c", subcore_axis_name="s",
      num_cores=NUM_CORES, num_subcores=NUM_SUBCORES)

  @functools.partial(
      pl.kernel,
      out_type=jax.ShapeDtypeStruct((n_total, OUT_D), jnp.float32),
      mesh=mesh,
      compiler_params=pltpu.CompilerParams(use_tc_tiling_on_sc=False),
      scratch_types=[
          pltpu.VMEM((idx_rows, IDX_MINOR), jnp.int32),
          pltpu.VMEM((idx_rows, IDX_MINOR), jnp.int32),
          pltpu.VMEM((2, CHUNK, TOKEN_EMB), jnp.float32),
          pltpu.VMEM((2, CHUNK, POS_EMB), jnp.float32),
          pltpu.SemaphoreType.DMA,
          pltpu.SemaphoreType.DMA,
          pltpu.SemaphoreType.DMA,
          pltpu.SemaphoreType.DMA,
      ],
  )
  def emb_kernel(tok_hbm, pos_hbm, tok_tab_hbm, pos_tab_hbm, out_hbm,
                 tok_idx_v, pos_idx_v, tok_rows_v, pos_rows_v,
                 sg0, sg1, sw0, sw1):
    wid = lax.axis_index("s") * NUM_CORES + lax.axis_index("c")
    w_el = wid * per_w
    sg = (sg0, sg1)
    sw = (sw0, sw1)

    row0 = wid * idx_rows
    pltpu.sync_copy(tok_hbm.at[pl.ds(row0, idx_rows)], tok_idx_v)
    pltpu.sync_copy(pos_hbm.at[pl.ds(row0, idx_rows)], pos_idx_v)

    def issue_gathers(g, p):
      for t in range(K):
        row = g * K + t
        pltpu.async_copy(
            tok_tab_hbm.at[tok_idx_v.at[row]],
            tok_rows_v.at[p].at[pl.ds(t * IDX_MINOR, IDX_MINOR)], sg[p])
        pltpu.async_copy(
            pos_tab_hbm.at[pos_idx_v.at[row]],
            pos_rows_v.at[p].at[pl.ds(t * IDX_MINOR, IDX_MINOR)], sg[p])

    def wait_gathers(p):
      for t in range(K):
        pltpu.make_async_copy(
            tok_tab_hbm.at[tok_idx_v.at[t]],
            tok_rows_v.at[p].at[pl.ds(t * IDX_MINOR, IDX_MINOR)],
            sg[p]).wait()
        pltpu.make_async_copy(
            pos_tab_hbm.at[pos_idx_v.at[t]],
            pos_rows_v.at[p].at[pl.ds(t * IDX_MINOR, IDX_MINOR)],
            sg[p]).wait()

    def issue_writes(g, p):
      base = w_el + g * CHUNK
      pltpu.async_copy(
          tok_rows_v.at[p],
          out_hbm.at[pl.ds(base, CHUNK), pl.ds(0, TOKEN_EMB)], sw[p])
      pltpu.async_copy(
          pos_rows_v.at[p],
          out_hbm.at[pl.ds(base, CHUNK), pl.ds(TOKEN_EMB, POS_EMB)], sw[p])

    def wait_writes(p):
      pltpu.make_async_copy(
          tok_rows_v.at[p],
          out_hbm.at[pl.ds(w_el, CHUNK), pl.ds(0, TOKEN_EMB)],
          sw[p]).wait()
      pltpu.make_async_copy(
          pos_rows_v.at[p],
          out_hbm.at[pl.ds(w_el, CHUNK), pl.ds(TOKEN_EMB, POS_EMB)],
          sw[p]).wait()

    issue_gathers(0, 0)
    issue_gathers(1, 1)
    wait_gathers(0)
    issue_writes(0, 0)

    @pl.loop(2, n_iters, step=2)
    def _steady(gi):
      for b in range(2):
        g = gi + b
        wait_writes(b)
        issue_gathers(g, b)
        wait_gathers(1 - b)
        issue_writes(g - 1, 1 - b)

    wait_gathers(1)
    issue_writes(n_iters - 1, 1)
    wait_writes(0)
    wait_writes(1)

  return emb_kernel


@jax.jit
def kernel(tokens, pos, token_table, pos_table):
  B, L = tokens.shape
  n_total = B * L
  emb = _make_kernel(n_total)
  out = emb(tokens.reshape(n_total // IDX_MINOR, IDX_MINOR),
            pos.reshape(n_total // IDX_MINOR, IDX_MINOR),
            token_table, pos_table)
  return out.reshape(B, L, OUT_D)
